# baseline (speedup 1.0000x reference)
import functools

import jax
import jax.numpy as jnp
from jax import lax
from jax.experimental import pallas as pl
from jax.experimental.pallas import tpu as pltpu
from jax.experimental.pallas import tpu_sc as plsc

VOCAB = 1000000
EMBED_DIM = 64
BATCH = 16384
HIST_LEN = 50

PAIR_ROWS = VOCAB // 2
CHUNK = 128
NCHUNK_TOTAL = BATCH * HIST_LEN // CHUNK
NUM_WORKERS = 32
NCHUNK = NCHUNK_TOTAL // NUM_WORKERS
BTILES = BATCH // CHUNK
NBUF = 4
LANES = 16


def _encode_kernel(table2, idx_hbm, out, idx_v, rows, planes, pidx,
                   gsems, osems):
    wid = lax.axis_index("s") * 2 + lax.axis_index("c")
    pltpu.sync_copy(idx_hbm.at[wid], idx_v)

    iota = lax.iota(jnp.int32, LANES)

    def fill_pidx(b, g):
        for j0 in range(CHUNK // LANES):
            iv = idx_v[g, pl.ds(j0 * LANES, LANES)]
            pidx[b][pl.ds(j0 * LANES, LANES)] = lax.shift_right_logical(iv, 1)

    def start_gather(b, g):
        fill_pidx(b, g)
        pltpu.async_copy(table2.at[pidx[b]], rows[b], gsems[b])

    def out_block(g):
        c = wid * NCHUNK + g
        t = c // BTILES
        bt = c - t * BTILES
        return out.at[t, :, pl.ds(bt * CHUNK, CHUNK)]

    def build_plane(b, g):
        hcols = []
        for j0 in range(CHUNK // LANES):
            iv = idx_v[g, pl.ds(j0 * LANES, LANES)]
            hcols.append(lax.shift_left(lax.bitwise_and(iv, 1), 6))

        @pl.loop(0, EMBED_DIM)
        def _per_feature(d):
            for j0 in range(CHUNK // LANES):
                col = hcols[j0] + d
                row = iota + (j0 * LANES)
                v = plsc.load_gather(rows[b], [row, col])
                planes[b][d, pl.ds(j0 * LANES, LANES)] = v

    for b in range(NBUF):
        start_gather(b, b)

    @pl.loop(0, NCHUNK, step=NBUF)
    def _body(g0):
        for b in range(NBUF):
            g = g0 + b

            @pl.when(g0 > 0)
            def _drain_prev_out():
                pltpu.make_async_copy(planes[b], out_block(g - NBUF),
                                      osems[b]).wait()

            pltpu.make_async_copy(table2.at[pidx[b]], rows[b],
                                  gsems[b]).wait()
            build_plane(b, g)
            pltpu.async_copy(planes[b], out_block(g), osems[b])

            @pl.when(g + NBUF < NCHUNK)
            def _refill():
                start_gather(b, g + NBUF)

    for b in range(NBUF):
        pltpu.make_async_copy(planes[b], out_block(NCHUNK - NBUF + b),
                              osems[b]).wait()


def kernel(item_embeddings, batch_data):
    table2 = item_embeddings.reshape(PAIR_ROWS, 2 * EMBED_DIM)
    idx = batch_data.astype(jnp.int32).T.reshape(NUM_WORKERS, NCHUNK, CHUNK)
    mesh = plsc.VectorSubcoreMesh(core_axis_name="c", subcore_axis_name="s")
    planes = pl.kernel(
        _encode_kernel,
        out_type=jax.ShapeDtypeStruct((HIST_LEN, EMBED_DIM, BATCH),
                                      jnp.float32),
        mesh=mesh,
        scratch_types=[
            pltpu.VMEM((NCHUNK, CHUNK), jnp.int32),
            tuple(pltpu.VMEM((CHUNK, 2 * EMBED_DIM), jnp.float32)
                  for _ in range(NBUF)),
            tuple(pltpu.VMEM((EMBED_DIM, CHUNK), jnp.float32)
                  for _ in range(NBUF)),
            tuple(pltpu.VMEM((CHUNK,), jnp.int32) for _ in range(NBUF)),
            tuple(pltpu.SemaphoreType.DMA for _ in range(NBUF)),
            tuple(pltpu.SemaphoreType.DMA for _ in range(NBUF)),
        ],
        compiler_params=pltpu.CompilerParams(use_tc_tiling_on_sc=True,
                                             needs_layout_passes=False),
    )(table2, idx)
    return planes.transpose(2, 0, 1)
